# Initial kernel scaffold; baseline (speedup 1.0000x reference)
import jax
import jax.numpy as jnp
from jax.experimental import pallas as pl


def kernel(x, edge_index, edge_attr, batch_idx, global_features, W_node, b_node, W_edge, b_edge, conv_W, conv_att, conv_bias, Wg1, bg1, Wg2, bg2, Wo1, bo1, Wo2, bo2):
    raise NotImplementedError("write your pallas kernel here")


# speedup vs baseline: 1.8532x; 1.8532x over previous
import functools

import jax
import jax.numpy as jnp
import numpy as np
from jax import lax
from jax.experimental import pallas as pl
from jax.experimental.pallas import tpu as pltpu
from jax.experimental.pallas import tpu_sc as plsc

N_NODES = 10000
E = 160000
F_NODE = 128
F_EDGE = 16
HID = 64
HEADS = 4
N_LAYERS = 3
N_GRAPHS = 64
G_FEAT = 108

Q = E // 4
BQ = 2000
NT = Q // BQ
BN_SCALE = 1.0 / np.sqrt(1.0 + 1e-3)
SLOPE = 0.2

NC = 2
NS = 16
NW = NC * NS
CHUNK = 128
GE = 2 * E
GE_PAD = 327680
CPW = GE_PAD // (NW * CHUNK)
NCHUNK = E // CHUNK
SCPW = 40
N_PAD = 10240
NPS = N_PAD // NS


def _leaky(x):
    return jnp.where(x > 0, x, SLOPE * x)


def _node_enc_body(x_ref, w_ref, b_ref, o_ref):
    o_ref[...] = _leaky(
        jnp.dot(x_ref[...], w_ref[...], preferred_element_type=jnp.float32)
        + b_ref[...]
    )


def _node_encoder(x, w, b):
    return pl.pallas_call(
        _node_enc_body,
        out_shape=jax.ShapeDtypeStruct((N_NODES, 2 * HID), jnp.float32),
    )(x, w, b)


@functools.cache
def _build_sc_gather():
    mesh = plsc.VectorSubcoreMesh(core_axis_name="c", subcore_axis_name="s")

    @functools.partial(
        pl.kernel,
        out_type=jax.ShapeDtypeStruct((GE_PAD, 2 * HID), jnp.float32),
        mesh=mesh,
        scratch_types=[
            pltpu.VMEM((CHUNK,), jnp.int32),
            pltpu.VMEM((CHUNK, 2 * HID), jnp.float32),
            pltpu.SemaphoreType.DMA,
        ],
    )
    def _sc_gather_k(h_hbm, idx_hbm, out_hbm, idx_v, rows_v, sem):
        wid = lax.axis_index("s") * NC + lax.axis_index("c")
        base = wid * (CHUNK * CPW)

        def body(j, carry):
            off = base + j * CHUNK
            pltpu.sync_copy(idx_hbm.at[pl.ds(off, CHUNK)], idx_v)
            pltpu.async_copy(h_hbm.at[idx_v], rows_v, sem).wait()
            pltpu.sync_copy(rows_v, out_hbm.at[pl.ds(off, CHUNK)])
            return carry

        lax.fori_loop(0, CPW, body, 0)

    return _sc_gather_k


def _sc_gather(h, gat_idx):
    return _build_sc_gather()(h, gat_idx)


@functools.cache
def _build_sc_scatter():
    mesh = plsc.VectorSubcoreMesh(core_axis_name="c", subcore_axis_name="s")

    @functools.partial(
        pl.kernel,
        out_type=jax.ShapeDtypeStruct((NC, N_PAD, 2 * HID), jnp.float32),
        mesh=mesh,
        scratch_types=[
            pltpu.VMEM((CHUNK,), jnp.int32),
            pltpu.VMEM((CHUNK, 2 * HID), jnp.float32),
            pltpu.VMEM_SHARED((N_PAD, 2 * HID), jnp.float32),
            pltpu.SemaphoreType.DMA,
        ],
    )
    def _sc_scatter_k(v_hbm, idx_hbm, zeros_hbm, out_hbm, idx_v, vals_v,
                      table, sem):
        cid = lax.axis_index("c")
        sid = lax.axis_index("s")
        wid = sid * NC + cid
        pltpu.sync_copy(
            zeros_hbm.at[pl.ds(sid * NPS, NPS)], table.at[pl.ds(sid * NPS, NPS)]
        )
        plsc.subcore_barrier()
        nj = jnp.minimum(SCPW, jnp.maximum(0, NCHUNK - wid * SCPW))

        def body(j, carry):
            chunk = wid * SCPW + j
            pltpu.sync_copy(idx_hbm.at[pl.ds(chunk * CHUNK, CHUNK)], idx_v)
            pltpu.sync_copy(v_hbm.at[pl.ds(chunk * CHUNK, CHUNK)], vals_v)
            pltpu.sync_copy(vals_v, table.at[idx_v], add=True)
            return carry

        lax.fori_loop(0, nj, body, 0)

        plsc.subcore_barrier()
        pltpu.sync_copy(
            table.at[pl.ds(sid * NPS, NPS)],
            out_hbm.at[cid, pl.ds(sid * NPS, NPS)],
        )

    return _sc_scatter_k


def _sc_scatter(v_flat, idx2d, zeros_tab):
    return _build_sc_scatter()(v_flat, idx2d, zeros_tab)


def _edge_body(hr0, hr1, hr2, hr3, hc0, hc1, hc2, hc3, ea0, ea1, ea2, ea3,
               we_ref, be_ref, w_ref, atti_ref, attj_ref, o_ref):
    hrs = (hr0, hr1, hr2, hr3)
    hcs = (hc0, hc1, hc2, hc3)
    eas = (ea0, ea1, ea2, ea3)
    we = we_ref[...]
    be = be_ref[...]
    w = w_ref[...]
    atti = atti_ref[...]
    attj = attj_ref[...]
    pooled = [None] * HEADS
    for c in range(4):
        e_c = _leaky(
            jnp.dot(eas[c][...], we, preferred_element_type=jnp.float32) + be
        )
        cat_i = jnp.concatenate([hrs[c][...][:, :HID], e_c], axis=1)
        cat_j = jnp.concatenate([hcs[c][...][:, :HID], e_c], axis=1)
        xi = _leaky(jnp.dot(cat_i, w, preferred_element_type=jnp.float32))
        xj = _leaky(jnp.dot(cat_j, w, preferred_element_type=jnp.float32))
        logits = []
        for hh in range(HEADS):
            li = jnp.sum(
                xi[:, hh * HID:(hh + 1) * HID] * atti[hh:hh + 1, :],
                axis=1, keepdims=True,
            )
            lj = jnp.sum(
                xj[:, hh * HID:(hh + 1) * HID] * attj[hh:hh + 1, :],
                axis=1, keepdims=True,
            )
            logits.append(_leaky(li + lj) * BN_SCALE)
        m = jnp.maximum(jnp.maximum(logits[0], logits[1]),
                        jnp.maximum(logits[2], logits[3]))
        exs = [jnp.exp(l - m) for l in logits]
        tot = exs[0] + exs[1] + exs[2] + exs[3]
        inv = 0.25 / tot
        for hh in range(HEADS):
            contrib = xj[:, hh * HID:(hh + 1) * HID] * (exs[hh] * inv)
            pooled[hh] = contrib if pooled[hh] is None else pooled[hh] + contrib
    for hh in range(HEADS):
        o_ref[hh] = jnp.concatenate(
            [pooled[hh], jnp.zeros_like(pooled[hh])], axis=1)


def _edge_layer(gathered, ea_perm, we, be, w, atti, attj):
    hb = (BQ, 2 * HID)
    spec_h = [
        pl.BlockSpec(hb, functools.partial(lambda c, t: (c * NT + t, 0), c))
        for c in range(4)
    ]
    spec_hc = [
        pl.BlockSpec(
            hb, functools.partial(lambda c, t: (4 * NT + c * NT + t, 0), c)
        )
        for c in range(4)
    ]
    spec_ea = [
        pl.BlockSpec(
            (BQ, F_EDGE), functools.partial(lambda c, t: (c * NT + t, 0), c)
        )
        for c in range(4)
    ]
    full = lambda shape: pl.BlockSpec(shape, lambda t: (0,) * len(shape))
    return pl.pallas_call(
        _edge_body,
        grid=(NT,),
        in_specs=spec_h + spec_hc + spec_ea + [
            full((F_EDGE, HID)), full((1, HID)), full((2 * HID, HEADS * HID)),
            full((HEADS, HID)), full((HEADS, HID)),
        ],
        out_specs=pl.BlockSpec((HEADS, BQ, 2 * HID), lambda t: (0, t, 0)),
        out_shape=jax.ShapeDtypeStruct((HEADS, Q, 2 * HID), jnp.float32),
    )(*([gathered] * 8), *([ea_perm] * 4), we, be, w, atti, attj)


def _combine0_body(p_ref, b_ref, o_ref):
    r = p_ref[0, :N_NODES, :HID] + p_ref[1, :N_NODES, :HID] + b_ref[...]
    o_ref[...] = jnp.concatenate([r, jnp.zeros_like(r)], axis=1)


def _combine_body(base_ref, p_ref, b_ref, o_ref):
    r = base_ref[...][:, :HID] + p_ref[0, :N_NODES, :HID] \
        + p_ref[1, :N_NODES, :HID] + b_ref[...]
    o_ref[...] = jnp.concatenate([r, jnp.zeros_like(r)], axis=1)


def _combine(partials, bias_row, base=None):
    shape = jax.ShapeDtypeStruct((N_NODES, 2 * HID), jnp.float32)
    if base is None:
        return pl.pallas_call(_combine0_body, out_shape=shape)(partials, bias_row)
    return pl.pallas_call(_combine_body, out_shape=shape)(base, partials, bias_row)


def _readout_body(h_ref, h0_ref, bid_ref, gf_ref, wg1b_ref, wg1a_ref, bg1_ref,
                  wg2_ref, bg2_ref, wo1_ref, bo1_ref, wo2_ref, bo2_ref, o_ref):
    h = h_ref[...][:, :HID] + h0_ref[...][:, :HID]
    bid = bid_ref[...]
    iota = lax.broadcasted_iota(jnp.int32, (N_NODES, N_GRAPHS), 1)
    oh = iota == bid
    ohf = oh.astype(jnp.float32)
    gfw = jnp.dot(gf_ref[...], wg1b_ref[...],
                  preferred_element_type=jnp.float32)
    a = _leaky(
        jnp.dot(h, wg1a_ref[...], preferred_element_type=jnp.float32)
        + jnp.dot(ohf, gfw, preferred_element_type=jnp.float32)
        + bg1_ref[...]
    )
    scores = jnp.dot(a, wg2_ref[...], preferred_element_type=jnp.float32) \
        + bg2_ref[...]
    masked = jnp.where(oh, scores, -1e30)
    m = jnp.max(masked, axis=0, keepdims=True)
    m_sel = jnp.sum(ohf * m, axis=1, keepdims=True)
    ex = jnp.exp(scores - m_sel)
    s = jnp.sum(ohf * ex, axis=0, keepdims=True)
    s_sel = jnp.sum(ohf * s, axis=1, keepdims=True)
    hw = h * (ex / s_sel)
    pooled = lax.dot_general(ohf, hw, (((0,), (0,)), ((), ())),
                             preferred_element_type=jnp.float32)
    z = jnp.maximum(
        jnp.dot(pooled, wo1_ref[...], preferred_element_type=jnp.float32)
        + bo1_ref[...], 0.0)
    o_ref[...] = jnp.dot(z, wo2_ref[...], preferred_element_type=jnp.float32) \
        + bo2_ref[...]


def _readout(h, h0, bid2d, gf, wg1b, wg1a, bg1, wg2, bg2, wo1, bo1, wo2, bo2):
    return pl.pallas_call(
        _readout_body,
        out_shape=jax.ShapeDtypeStruct((N_GRAPHS, 1), jnp.float32),
    )(h, h0, bid2d, gf, wg1b, wg1a, bg1, wg2, bg2, wo1, bo1, wo2, bo2)


def kernel(x, edge_index, edge_attr, batch_idx, global_features, W_node,
           b_node, W_edge, b_edge, conv_W, conv_att, conv_bias, Wg1, bg1,
           Wg2, bg2, Wo1, bo1, Wo2, bo2):
    row = edge_index[0]
    col = edge_index[1]
    row_p = row.reshape(Q, 4).T.reshape(-1)
    col_p = col.reshape(Q, 4).T.reshape(-1)
    gat_idx = jnp.concatenate(
        [row_p, col_p, jnp.zeros((GE_PAD - GE,), jnp.int32)])
    idx_flat = jnp.pad(row, (0, NW * SCPW * CHUNK - E))
    ea_perm = edge_attr.reshape(Q, 4, F_EDGE).transpose(1, 0, 2) \
        .reshape(4 * Q, F_EDGE)
    zeros_tab = jnp.zeros((N_PAD, 2 * HID), jnp.float32)
    bid2d = batch_idx.reshape(N_NODES, 1)
    be_row = b_edge.reshape(1, HID)

    h0 = _node_encoder(x, jnp.pad(W_node, ((0, 0), (0, HID))),
                       jnp.pad(b_node, (0, HID)).reshape(1, 2 * HID))
    h = h0
    for i in range(N_LAYERS):
        gathered = _sc_gather(h, gat_idx)
        v = _edge_layer(
            gathered, ea_perm, W_edge, be_row, conv_W[i],
            conv_att[i, 0, :, :HID], conv_att[i, 0, :, HID:])
        partials = _sc_scatter(v.reshape(E, 2 * HID), idx_flat, zeros_tab)
        h = _combine(partials, conv_bias[i].reshape(1, HID),
                     base=None if i == 0 else h)

    out = _readout(
        h, h0, bid2d, global_features, Wg1[HID:], Wg1[:HID],
        bg1.reshape(1, HID), Wg2, bg2.reshape(1, 1), Wo1,
        bo1.reshape(1, HID), Wo2, bo2.reshape(1, 1))
    return out[:, 0]


# speedup vs baseline: 2.0531x; 1.1079x over previous
import functools

import jax
import jax.numpy as jnp
import numpy as np
from jax import lax
from jax.experimental import pallas as pl
from jax.experimental.pallas import tpu as pltpu
from jax.experimental.pallas import tpu_sc as plsc

N_NODES = 10000
E = 160000
F_NODE = 128
F_EDGE = 16
HID = 64
HEADS = 4
N_LAYERS = 3
N_GRAPHS = 64
G_FEAT = 108

Q = E // 4
BQ = 2000
NT = Q // BQ
BN_SCALE = 1.0 / np.sqrt(1.0 + 1e-3)
SLOPE = 0.2

NC = 2
NS = 16
NW = NC * NS
CHUNK = 128
KBUF = 4
KBUF_S = 2
GE = 2 * E
GE_PAD = 327680
CPW = GE_PAD // (NW * CHUNK)
NCHUNK = E // CHUNK
SCPW = 40
N_PAD = 10240
NPS = N_PAD // NS


def _leaky(x):
    return jnp.where(x > 0, x, SLOPE * x)


def _node_enc_body(x_ref, w_ref, b_ref, o_ref):
    o_ref[...] = _leaky(
        jnp.dot(x_ref[...], w_ref[...], preferred_element_type=jnp.float32)
        + b_ref[...]
    )


def _node_encoder(x, w, b):
    return pl.pallas_call(
        _node_enc_body,
        out_shape=jax.ShapeDtypeStruct((N_NODES, 2 * HID), jnp.float32),
    )(x, w, b)


@functools.cache
def _build_sc_gather():
    mesh = plsc.VectorSubcoreMesh(core_axis_name="c", subcore_axis_name="s")

    @functools.partial(
        pl.kernel,
        out_type=jax.ShapeDtypeStruct((GE_PAD, 2 * HID), jnp.float32),
        mesh=mesh,
        scratch_types=[
            pltpu.VMEM((CHUNK * CPW,), jnp.int32),
            [pltpu.VMEM((CHUNK, 2 * HID), jnp.float32) for _ in range(KBUF)],
            pltpu.SemaphoreType.DMA,
            pltpu.SemaphoreType.DMA,
        ],
    )
    def _sc_gather_k(h_hbm, idx_hbm, out_hbm, idx_v, rows, sem_g, sem_w):
        wid = lax.axis_index("s") * NC + lax.axis_index("c")
        base = wid * (CHUNK * CPW)
        pltpu.sync_copy(idx_hbm.at[pl.ds(base, CHUNK * CPW)], idx_v)

        def body(j, carry):
            gs = [
                pltpu.async_copy(
                    h_hbm.at[idx_v.at[pl.ds((j * KBUF + b) * CHUNK, CHUNK)]],
                    rows[b], sem_g)
                for b in range(KBUF)
            ]
            for g in gs:
                g.wait()
            ws = [
                pltpu.async_copy(
                    rows[b],
                    out_hbm.at[pl.ds(base + (j * KBUF + b) * CHUNK, CHUNK)],
                    sem_w)
                for b in range(KBUF)
            ]
            for w in ws:
                w.wait()
            return carry

        lax.fori_loop(0, CPW // KBUF, body, 0)

    return _sc_gather_k


def _sc_gather(h, gat_idx):
    return _build_sc_gather()(h, gat_idx)


@functools.cache
def _build_sc_scatter():
    mesh = plsc.VectorSubcoreMesh(core_axis_name="c", subcore_axis_name="s")

    @functools.partial(
        pl.kernel,
        out_type=jax.ShapeDtypeStruct((NC, N_PAD, 2 * HID), jnp.float32),
        mesh=mesh,
        scratch_types=[
            pltpu.VMEM((CHUNK * SCPW,), jnp.int32),
            [pltpu.VMEM((CHUNK, 2 * HID), jnp.float32) for _ in range(KBUF_S)],
            pltpu.VMEM_SHARED((N_PAD, 2 * HID), jnp.float32),
            pltpu.SemaphoreType.DMA,
            pltpu.SemaphoreType.DMA,
        ],
    )
    def _sc_scatter_k(v_hbm, idx_hbm, zeros_hbm, out_hbm, idx_v, vals,
                      table, sem_v, sem_s):
        cid = lax.axis_index("c")
        sid = lax.axis_index("s")
        wid = sid * NC + cid
        pltpu.sync_copy(
            zeros_hbm.at[pl.ds(sid * NPS, NPS)], table.at[pl.ds(sid * NPS, NPS)]
        )
        plsc.subcore_barrier()
        nj = jnp.minimum(SCPW, jnp.maximum(0, NCHUNK - wid * SCPW))
        base = wid * SCPW * CHUNK
        pltpu.sync_copy(idx_hbm.at[pl.ds(base, SCPW * CHUNK)], idx_v)

        def group_body(j, carry):
            ls = [
                pltpu.async_copy(
                    v_hbm.at[pl.ds(base + (j * KBUF_S + b) * CHUNK, CHUNK)],
                    vals[b], sem_v)
                for b in range(KBUF_S)
            ]
            for l in ls:
                l.wait()
            ss = [
                pltpu.async_copy(
                    vals[b],
                    table.at[idx_v.at[pl.ds((j * KBUF_S + b) * CHUNK, CHUNK)]],
                    sem_s, add=True)
                for b in range(KBUF_S)
            ]
            for s in ss:
                s.wait()
            return carry

        lax.fori_loop(0, nj // KBUF_S, group_body, 0)

        def tail_body(j, carry):
            pltpu.async_copy(
                v_hbm.at[pl.ds(base + j * CHUNK, CHUNK)], vals[0], sem_v
            ).wait()
            pltpu.async_copy(
                vals[0], table.at[idx_v.at[pl.ds(j * CHUNK, CHUNK)]],
                sem_s, add=True
            ).wait()
            return carry

        lax.fori_loop((nj // KBUF_S) * KBUF_S, nj, tail_body, 0)

        plsc.subcore_barrier()
        pltpu.sync_copy(
            table.at[pl.ds(sid * NPS, NPS)],
            out_hbm.at[cid, pl.ds(sid * NPS, NPS)],
        )

    return _sc_scatter_k


def _sc_scatter(v_flat, idx2d, zeros_tab):
    return _build_sc_scatter()(v_flat, idx2d, zeros_tab)


def _edge_body(hr0, hr1, hr2, hr3, hc0, hc1, hc2, hc3, ea0, ea1, ea2, ea3,
               we_ref, be_ref, w_ref, atti_ref, attj_ref, o_ref):
    hrs = (hr0, hr1, hr2, hr3)
    hcs = (hc0, hc1, hc2, hc3)
    eas = (ea0, ea1, ea2, ea3)
    we = we_ref[...]
    be = be_ref[...]
    w = w_ref[...]
    atti = atti_ref[...]
    attj = attj_ref[...]
    pooled = [None] * HEADS
    for c in range(4):
        e_c = _leaky(
            jnp.dot(eas[c][...], we, preferred_element_type=jnp.float32) + be
        )
        cat_i = jnp.concatenate([hrs[c][...][:, :HID], e_c], axis=1)
        cat_j = jnp.concatenate([hcs[c][...][:, :HID], e_c], axis=1)
        xi = _leaky(jnp.dot(cat_i, w, preferred_element_type=jnp.float32))
        xj = _leaky(jnp.dot(cat_j, w, preferred_element_type=jnp.float32))
        logits = []
        for hh in range(HEADS):
            li = jnp.sum(
                xi[:, hh * HID:(hh + 1) * HID] * atti[hh:hh + 1, :],
                axis=1, keepdims=True,
            )
            lj = jnp.sum(
                xj[:, hh * HID:(hh + 1) * HID] * attj[hh:hh + 1, :],
                axis=1, keepdims=True,
            )
            logits.append(_leaky(li + lj) * BN_SCALE)
        m = jnp.maximum(jnp.maximum(logits[0], logits[1]),
                        jnp.maximum(logits[2], logits[3]))
        exs = [jnp.exp(l - m) for l in logits]
        tot = exs[0] + exs[1] + exs[2] + exs[3]
        inv = 0.25 / tot
        for hh in range(HEADS):
            contrib = xj[:, hh * HID:(hh + 1) * HID] * (exs[hh] * inv)
            pooled[hh] = contrib if pooled[hh] is None else pooled[hh] + contrib
    for hh in range(HEADS):
        o_ref[hh] = jnp.concatenate(
            [pooled[hh], jnp.zeros_like(pooled[hh])], axis=1)


def _edge_layer(gathered, ea_perm, we, be, w, atti, attj):
    hb = (BQ, 2 * HID)
    spec_h = [
        pl.BlockSpec(hb, functools.partial(lambda c, t: (c * NT + t, 0), c))
        for c in range(4)
    ]
    spec_hc = [
        pl.BlockSpec(
            hb, functools.partial(lambda c, t: (4 * NT + c * NT + t, 0), c)
        )
        for c in range(4)
    ]
    spec_ea = [
        pl.BlockSpec(
            (BQ, F_EDGE), functools.partial(lambda c, t: (c * NT + t, 0), c)
        )
        for c in range(4)
    ]
    full = lambda shape: pl.BlockSpec(shape, lambda t: (0,) * len(shape))
    return pl.pallas_call(
        _edge_body,
        grid=(NT,),
        in_specs=spec_h + spec_hc + spec_ea + [
            full((F_EDGE, HID)), full((1, HID)), full((2 * HID, HEADS * HID)),
            full((HEADS, HID)), full((HEADS, HID)),
        ],
        out_specs=pl.BlockSpec((HEADS, BQ, 2 * HID), lambda t: (0, t, 0)),
        out_shape=jax.ShapeDtypeStruct((HEADS, Q, 2 * HID), jnp.float32),
    )(*([gathered] * 8), *([ea_perm] * 4), we, be, w, atti, attj)


def _combine0_body(p_ref, b_ref, o_ref):
    r = p_ref[0, :N_NODES, :HID] + p_ref[1, :N_NODES, :HID] + b_ref[...]
    o_ref[...] = jnp.concatenate([r, jnp.zeros_like(r)], axis=1)


def _combine_body(base_ref, p_ref, b_ref, o_ref):
    r = base_ref[...][:, :HID] + p_ref[0, :N_NODES, :HID] \
        + p_ref[1, :N_NODES, :HID] + b_ref[...]
    o_ref[...] = jnp.concatenate([r, jnp.zeros_like(r)], axis=1)


def _combine(partials, bias_row, base=None):
    shape = jax.ShapeDtypeStruct((N_NODES, 2 * HID), jnp.float32)
    if base is None:
        return pl.pallas_call(_combine0_body, out_shape=shape)(partials, bias_row)
    return pl.pallas_call(_combine_body, out_shape=shape)(base, partials, bias_row)


def _readout_body(h_ref, h0_ref, bid_ref, gf_ref, wg1b_ref, wg1a_ref, bg1_ref,
                  wg2_ref, bg2_ref, wo1_ref, bo1_ref, wo2_ref, bo2_ref, o_ref):
    h = h_ref[...][:, :HID] + h0_ref[...][:, :HID]
    bid = bid_ref[...]
    iota = lax.broadcasted_iota(jnp.int32, (N_NODES, N_GRAPHS), 1)
    oh = iota == bid
    ohf = oh.astype(jnp.float32)
    gfw = jnp.dot(gf_ref[...], wg1b_ref[...],
                  preferred_element_type=jnp.float32)
    a = _leaky(
        jnp.dot(h, wg1a_ref[...], preferred_element_type=jnp.float32)
        + jnp.dot(ohf, gfw, preferred_element_type=jnp.float32)
        + bg1_ref[...]
    )
    scores = jnp.dot(a, wg2_ref[...], preferred_element_type=jnp.float32) \
        + bg2_ref[...]
    masked = jnp.where(oh, scores, -1e30)
    m = jnp.max(masked, axis=0, keepdims=True)
    m_sel = jnp.sum(ohf * m, axis=1, keepdims=True)
    ex = jnp.exp(scores - m_sel)
    s = jnp.sum(ohf * ex, axis=0, keepdims=True)
    s_sel = jnp.sum(ohf * s, axis=1, keepdims=True)
    hw = h * (ex / s_sel)
    pooled = lax.dot_general(ohf, hw, (((0,), (0,)), ((), ())),
                             preferred_element_type=jnp.float32)
    z = jnp.maximum(
        jnp.dot(pooled, wo1_ref[...], preferred_element_type=jnp.float32)
        + bo1_ref[...], 0.0)
    o_ref[...] = jnp.dot(z, wo2_ref[...], preferred_element_type=jnp.float32) \
        + bo2_ref[...]


def _readout(h, h0, bid2d, gf, wg1b, wg1a, bg1, wg2, bg2, wo1, bo1, wo2, bo2):
    return pl.pallas_call(
        _readout_body,
        out_shape=jax.ShapeDtypeStruct((N_GRAPHS, 1), jnp.float32),
    )(h, h0, bid2d, gf, wg1b, wg1a, bg1, wg2, bg2, wo1, bo1, wo2, bo2)


def kernel(x, edge_index, edge_attr, batch_idx, global_features, W_node,
           b_node, W_edge, b_edge, conv_W, conv_att, conv_bias, Wg1, bg1,
           Wg2, bg2, Wo1, bo1, Wo2, bo2):
    row = edge_index[0]
    col = edge_index[1]
    row_p = row.reshape(Q, 4).T.reshape(-1)
    col_p = col.reshape(Q, 4).T.reshape(-1)
    gat_idx = jnp.concatenate(
        [row_p, col_p, jnp.zeros((GE_PAD - GE,), jnp.int32)])
    idx_flat = jnp.pad(row, (0, NW * SCPW * CHUNK - E))
    ea_perm = edge_attr.reshape(Q, 4, F_EDGE).transpose(1, 0, 2) \
        .reshape(4 * Q, F_EDGE)
    zeros_tab = jnp.zeros((N_PAD, 2 * HID), jnp.float32)
    bid2d = batch_idx.reshape(N_NODES, 1)
    be_row = b_edge.reshape(1, HID)

    h0 = _node_encoder(x, jnp.pad(W_node, ((0, 0), (0, HID))),
                       jnp.pad(b_node, (0, HID)).reshape(1, 2 * HID))
    h = h0
    for i in range(N_LAYERS):
        gathered = _sc_gather(h, gat_idx)
        v = _edge_layer(
            gathered, ea_perm, W_edge, be_row, conv_W[i],
            conv_att[i, 0, :, :HID], conv_att[i, 0, :, HID:])
        partials = _sc_scatter(v.reshape(E, 2 * HID), idx_flat, zeros_tab)
        h = _combine(partials, conv_bias[i].reshape(1, HID),
                     base=None if i == 0 else h)

    out = _readout(
        h, h0, bid2d, global_features, Wg1[HID:], Wg1[:HID],
        bg1.reshape(1, HID), Wg2, bg2.reshape(1, 1), Wo1,
        bo1.reshape(1, HID), Wo2, bo2.reshape(1, 1))
    return out[:, 0]


# speedup vs baseline: 3.2100x; 1.5635x over previous
import functools

import jax
import jax.numpy as jnp
import numpy as np
from jax import lax
from jax.experimental import pallas as pl
from jax.experimental.pallas import tpu as pltpu
from jax.experimental.pallas import tpu_sc as plsc

N_NODES = 10000
E = 160000
F_NODE = 128
F_EDGE = 16
HID = 64
HEADS = 4
N_LAYERS = 3
N_GRAPHS = 64
G_FEAT = 108

Q = E // 4
BQ = 2000
NT = Q // BQ
BN_SCALE = 1.0 / np.sqrt(1.0 + 1e-3)
SLOPE = 0.2

NC = 2
NS = 16
NW = NC * NS
CHUNK = 128
KBUF = 4
KBUF_S = 2
GE = 2 * E
GE_PAD = 327680
CPW = GE_PAD // (NW * CHUNK)
NCHUNK = E // CHUNK
SCPW = 40
N_PAD = 10240
NPS = N_PAD // NS


def _leaky(x):
    return jnp.where(x > 0, x, SLOPE * x)


def _node_enc_body(x_ref, w_ref, b_ref, o_ref):
    o_ref[...] = _leaky(
        jnp.dot(x_ref[...], w_ref[...], preferred_element_type=jnp.float32)
        + b_ref[...]
    )


def _node_encoder(x, w, b):
    return pl.pallas_call(
        _node_enc_body,
        out_shape=jax.ShapeDtypeStruct((N_NODES, 2 * HID), jnp.float32),
    )(x, w, b)


@functools.cache
def _build_sc_gather():
    mesh = plsc.VectorSubcoreMesh(core_axis_name="c", subcore_axis_name="s")

    @functools.partial(
        pl.kernel,
        out_type=jax.ShapeDtypeStruct((GE_PAD, 2 * HID), jnp.float32),
        mesh=mesh,
        scratch_types=[
            pltpu.VMEM((CHUNK * CPW,), jnp.int32),
            [pltpu.VMEM((CHUNK, 2 * HID), jnp.float32) for _ in range(KBUF_S)],
            pltpu.VMEM_SHARED((N_PAD, 2 * HID), jnp.float32),
            pltpu.SemaphoreType.DMA,
            pltpu.SemaphoreType.DMA,
        ],
    )
    def _sc_gather_k(h_hbm, idx_hbm, out_hbm, idx_v, rows, table, sem_g,
                     sem_w):
        sid = lax.axis_index("s")
        wid = sid * NC + lax.axis_index("c")
        base = wid * (CHUNK * CPW)
        @pl.when(sid < NS - 1)
        def _():
            pltpu.sync_copy(h_hbm.at[pl.ds(sid * NPS, NPS)],
                            table.at[pl.ds(sid * NPS, NPS)])

        @pl.when(sid == NS - 1)
        def _():
            pltpu.sync_copy(h_hbm.at[pl.ds((NS - 1) * NPS, N_NODES - (NS - 1) * NPS)],
                            table.at[pl.ds((NS - 1) * NPS, N_NODES - (NS - 1) * NPS)])

        pltpu.sync_copy(idx_hbm.at[pl.ds(base, CHUNK * CPW)], idx_v)
        plsc.subcore_barrier()

        def body(j, carry):
            gs = [
                pltpu.async_copy(
                    table.at[idx_v.at[pl.ds((j * KBUF_S + b) * CHUNK, CHUNK)]],
                    rows[b], sem_g)
                for b in range(KBUF_S)
            ]
            for g in gs:
                g.wait()
            ws = [
                pltpu.async_copy(
                    rows[b],
                    out_hbm.at[pl.ds(base + (j * KBUF_S + b) * CHUNK, CHUNK)],
                    sem_w)
                for b in range(KBUF_S)
            ]
            for w in ws:
                w.wait()
            return carry

        lax.fori_loop(0, CPW // KBUF_S, body, 0)

    return _sc_gather_k


def _sc_gather(h, gat_idx):
    return _build_sc_gather()(h, gat_idx)


@functools.cache
def _build_sc_scatter():
    mesh = plsc.VectorSubcoreMesh(core_axis_name="c", subcore_axis_name="s")

    @functools.partial(
        pl.kernel,
        out_type=jax.ShapeDtypeStruct((NC, N_PAD, 2 * HID), jnp.float32),
        mesh=mesh,
        scratch_types=[
            pltpu.VMEM((CHUNK * SCPW,), jnp.int32),
            [pltpu.VMEM((CHUNK, 2 * HID), jnp.float32) for _ in range(KBUF_S)],
            pltpu.VMEM_SHARED((N_PAD, 2 * HID), jnp.float32),
            pltpu.SemaphoreType.DMA,
            pltpu.SemaphoreType.DMA,
        ],
    )
    def _sc_scatter_k(v_hbm, idx_hbm, zeros_hbm, out_hbm, idx_v, vals,
                      table, sem_v, sem_s):
        cid = lax.axis_index("c")
        sid = lax.axis_index("s")
        wid = sid * NC + cid
        pltpu.sync_copy(
            zeros_hbm.at[pl.ds(sid * NPS, NPS)], table.at[pl.ds(sid * NPS, NPS)]
        )
        plsc.subcore_barrier()
        nj = jnp.minimum(SCPW, jnp.maximum(0, NCHUNK - wid * SCPW))
        base = wid * SCPW * CHUNK
        pltpu.sync_copy(idx_hbm.at[pl.ds(base, SCPW * CHUNK)], idx_v)

        def group_body(j, carry):
            ls = [
                pltpu.async_copy(
                    v_hbm.at[pl.ds(base + (j * KBUF_S + b) * CHUNK, CHUNK)],
                    vals[b], sem_v)
                for b in range(KBUF_S)
            ]
            for l in ls:
                l.wait()
            ss = [
                pltpu.async_copy(
                    vals[b],
                    table.at[idx_v.at[pl.ds((j * KBUF_S + b) * CHUNK, CHUNK)]],
                    sem_s, add=True)
                for b in range(KBUF_S)
            ]
            for s in ss:
                s.wait()
            return carry

        lax.fori_loop(0, nj // KBUF_S, group_body, 0)

        def tail_body(j, carry):
            pltpu.async_copy(
                v_hbm.at[pl.ds(base + j * CHUNK, CHUNK)], vals[0], sem_v
            ).wait()
            pltpu.async_copy(
                vals[0], table.at[idx_v.at[pl.ds(j * CHUNK, CHUNK)]],
                sem_s, add=True
            ).wait()
            return carry

        lax.fori_loop((nj // KBUF_S) * KBUF_S, nj, tail_body, 0)

        plsc.subcore_barrier()
        pltpu.sync_copy(
            table.at[pl.ds(sid * NPS, NPS)],
            out_hbm.at[cid, pl.ds(sid * NPS, NPS)],
        )

    return _sc_scatter_k


def _sc_scatter(v_flat, idx2d, zeros_tab):
    return _build_sc_scatter()(v_flat, idx2d, zeros_tab)


def _edge_body(hr0, hr1, hr2, hr3, hc0, hc1, hc2, hc3, ea0, ea1, ea2, ea3,
               we_ref, be_ref, w_ref, atti_ref, attj_ref, o_ref):
    hrs = (hr0, hr1, hr2, hr3)
    hcs = (hc0, hc1, hc2, hc3)
    eas = (ea0, ea1, ea2, ea3)
    we = we_ref[...]
    be = be_ref[...]
    w = w_ref[...]
    atti = atti_ref[...]
    attj = attj_ref[...]
    pooled = [None] * HEADS
    for c in range(4):
        e_c = _leaky(
            jnp.dot(eas[c][...], we, preferred_element_type=jnp.float32) + be
        )
        cat_i = jnp.concatenate([hrs[c][...][:, :HID], e_c], axis=1)
        cat_j = jnp.concatenate([hcs[c][...][:, :HID], e_c], axis=1)
        xi = _leaky(jnp.dot(cat_i, w, preferred_element_type=jnp.float32))
        xj = _leaky(jnp.dot(cat_j, w, preferred_element_type=jnp.float32))
        logits = []
        for hh in range(HEADS):
            li = jnp.sum(
                xi[:, hh * HID:(hh + 1) * HID] * atti[hh:hh + 1, :],
                axis=1, keepdims=True,
            )
            lj = jnp.sum(
                xj[:, hh * HID:(hh + 1) * HID] * attj[hh:hh + 1, :],
                axis=1, keepdims=True,
            )
            logits.append(_leaky(li + lj) * BN_SCALE)
        m = jnp.maximum(jnp.maximum(logits[0], logits[1]),
                        jnp.maximum(logits[2], logits[3]))
        exs = [jnp.exp(l - m) for l in logits]
        tot = exs[0] + exs[1] + exs[2] + exs[3]
        inv = 0.25 / tot
        for hh in range(HEADS):
            contrib = xj[:, hh * HID:(hh + 1) * HID] * (exs[hh] * inv)
            pooled[hh] = contrib if pooled[hh] is None else pooled[hh] + contrib
    for hh in range(HEADS):
        o_ref[hh] = jnp.concatenate(
            [pooled[hh], jnp.zeros_like(pooled[hh])], axis=1)


def _edge_layer(gathered, ea_perm, we, be, w, atti, attj):
    hb = (BQ, 2 * HID)
    spec_h = [
        pl.BlockSpec(hb, functools.partial(lambda c, t: (c * NT + t, 0), c))
        for c in range(4)
    ]
    spec_hc = [
        pl.BlockSpec(
            hb, functools.partial(lambda c, t: (4 * NT + c * NT + t, 0), c)
        )
        for c in range(4)
    ]
    spec_ea = [
        pl.BlockSpec(
            (BQ, F_EDGE), functools.partial(lambda c, t: (c * NT + t, 0), c)
        )
        for c in range(4)
    ]
    full = lambda shape: pl.BlockSpec(shape, lambda t: (0,) * len(shape))
    return pl.pallas_call(
        _edge_body,
        grid=(NT,),
        in_specs=spec_h + spec_hc + spec_ea + [
            full((F_EDGE, HID)), full((1, HID)), full((2 * HID, HEADS * HID)),
            full((HEADS, HID)), full((HEADS, HID)),
        ],
        out_specs=pl.BlockSpec((HEADS, BQ, 2 * HID), lambda t: (0, t, 0)),
        out_shape=jax.ShapeDtypeStruct((HEADS, Q, 2 * HID), jnp.float32),
    )(*([gathered] * 8), *([ea_perm] * 4), we, be, w, atti, attj)


def _combine0_body(p_ref, b_ref, o_ref):
    r = p_ref[0, :N_NODES, :HID] + p_ref[1, :N_NODES, :HID] + b_ref[...]
    o_ref[...] = jnp.concatenate([r, jnp.zeros_like(r)], axis=1)


def _combine_body(base_ref, p_ref, b_ref, o_ref):
    r = base_ref[...][:, :HID] + p_ref[0, :N_NODES, :HID] \
        + p_ref[1, :N_NODES, :HID] + b_ref[...]
    o_ref[...] = jnp.concatenate([r, jnp.zeros_like(r)], axis=1)


def _combine(partials, bias_row, base=None):
    shape = jax.ShapeDtypeStruct((N_NODES, 2 * HID), jnp.float32)
    if base is None:
        return pl.pallas_call(_combine0_body, out_shape=shape)(partials, bias_row)
    return pl.pallas_call(_combine_body, out_shape=shape)(base, partials, bias_row)


def _readout_body(h_ref, h0_ref, bid_ref, gf_ref, wg1b_ref, wg1a_ref, bg1_ref,
                  wg2_ref, bg2_ref, wo1_ref, bo1_ref, wo2_ref, bo2_ref, o_ref):
    h = h_ref[...][:, :HID] + h0_ref[...][:, :HID]
    bid = bid_ref[...]
    iota = lax.broadcasted_iota(jnp.int32, (N_NODES, N_GRAPHS), 1)
    oh = iota == bid
    ohf = oh.astype(jnp.float32)
    gfw = jnp.dot(gf_ref[...], wg1b_ref[...],
                  preferred_element_type=jnp.float32)
    a = _leaky(
        jnp.dot(h, wg1a_ref[...], preferred_element_type=jnp.float32)
        + jnp.dot(ohf, gfw, preferred_element_type=jnp.float32)
        + bg1_ref[...]
    )
    scores = jnp.dot(a, wg2_ref[...], preferred_element_type=jnp.float32) \
        + bg2_ref[...]
    masked = jnp.where(oh, scores, -1e30)
    m = jnp.max(masked, axis=0, keepdims=True)
    m_sel = jnp.sum(ohf * m, axis=1, keepdims=True)
    ex = jnp.exp(scores - m_sel)
    s = jnp.sum(ohf * ex, axis=0, keepdims=True)
    s_sel = jnp.sum(ohf * s, axis=1, keepdims=True)
    hw = h * (ex / s_sel)
    pooled = lax.dot_general(ohf, hw, (((0,), (0,)), ((), ())),
                             preferred_element_type=jnp.float32)
    z = jnp.maximum(
        jnp.dot(pooled, wo1_ref[...], preferred_element_type=jnp.float32)
        + bo1_ref[...], 0.0)
    o_ref[...] = jnp.dot(z, wo2_ref[...], preferred_element_type=jnp.float32) \
        + bo2_ref[...]


def _readout(h, h0, bid2d, gf, wg1b, wg1a, bg1, wg2, bg2, wo1, bo1, wo2, bo2):
    return pl.pallas_call(
        _readout_body,
        out_shape=jax.ShapeDtypeStruct((N_GRAPHS, 1), jnp.float32),
    )(h, h0, bid2d, gf, wg1b, wg1a, bg1, wg2, bg2, wo1, bo1, wo2, bo2)


def kernel(x, edge_index, edge_attr, batch_idx, global_features, W_node,
           b_node, W_edge, b_edge, conv_W, conv_att, conv_bias, Wg1, bg1,
           Wg2, bg2, Wo1, bo1, Wo2, bo2):
    row = edge_index[0]
    col = edge_index[1]
    row_p = row.reshape(Q, 4).T.reshape(-1)
    col_p = col.reshape(Q, 4).T.reshape(-1)
    gat_idx = jnp.concatenate(
        [row_p, col_p, jnp.zeros((GE_PAD - GE,), jnp.int32)])
    idx_flat = jnp.pad(row, (0, NW * SCPW * CHUNK - E))
    ea_perm = edge_attr.reshape(Q, 4, F_EDGE).transpose(1, 0, 2) \
        .reshape(4 * Q, F_EDGE)
    zeros_tab = jnp.zeros((N_PAD, 2 * HID), jnp.float32)
    bid2d = batch_idx.reshape(N_NODES, 1)
    be_row = b_edge.reshape(1, HID)

    h0 = _node_encoder(x, jnp.pad(W_node, ((0, 0), (0, HID))),
                       jnp.pad(b_node, (0, HID)).reshape(1, 2 * HID))
    h = h0
    for i in range(N_LAYERS):
        gathered = _sc_gather(h, gat_idx)
        v = _edge_layer(
            gathered, ea_perm, W_edge, be_row, conv_W[i],
            conv_att[i, 0, :, :HID], conv_att[i, 0, :, HID:])
        partials = _sc_scatter(v.reshape(E, 2 * HID), idx_flat, zeros_tab)
        h = _combine(partials, conv_bias[i].reshape(1, HID),
                     base=None if i == 0 else h)

    out = _readout(
        h, h0, bid2d, global_features, Wg1[HID:], Wg1[:HID],
        bg1.reshape(1, HID), Wg2, bg2.reshape(1, 1), Wo1,
        bo1.reshape(1, HID), Wo2, bo2.reshape(1, 1))
    return out[:, 0]


# speedup vs baseline: 4.2974x; 1.3387x over previous
import functools

import jax
import jax.numpy as jnp
import numpy as np
from jax import lax
from jax.experimental import pallas as pl
from jax.experimental.pallas import tpu as pltpu
from jax.experimental.pallas import tpu_sc as plsc

N_NODES = 10000
E = 160000
F_NODE = 128
F_EDGE = 16
HID = 64
HEADS = 4
N_LAYERS = 3
N_GRAPHS = 64
G_FEAT = 108

Q = E // 4
BQ = 2000
NT = Q // BQ
BN_SCALE = 1.0 / np.sqrt(1.0 + 1e-3)
LK_A = (1.0 + 0.2) / 2.0
LK_B = (1.0 - 0.2) / 2.0
SLOPE = 0.2

NC = 2
NS = 16
NW = NC * NS
CHUNK = 128
KBUF = 4
KBUF_S = 2
GE = 2 * E
GE_PAD = 327680
CPW = GE_PAD // (NW * CHUNK)
NCHUNK = E // CHUNK
SCPW = 40
N_PAD = 10240
NPS = N_PAD // NS


def _leaky(x):
    return jnp.maximum(x, SLOPE * x)


def _node_enc_body(x_ref, w_ref, b_ref, o_ref):
    o_ref[...] = _leaky(
        jnp.dot(x_ref[...], w_ref[...], preferred_element_type=jnp.float32)
        + b_ref[...]
    )


def _node_encoder(x, w, b):
    return pl.pallas_call(
        _node_enc_body,
        out_shape=jax.ShapeDtypeStruct((N_NODES, 2 * HID), jnp.float32),
    )(x, w, b)


@functools.cache
def _build_sc_gather():
    mesh = plsc.VectorSubcoreMesh(core_axis_name="c", subcore_axis_name="s")

    @functools.partial(
        pl.kernel,
        out_type=jax.ShapeDtypeStruct((GE_PAD, 2 * HID), jnp.float32),
        mesh=mesh,
        scratch_types=[
            pltpu.VMEM((CHUNK * CPW,), jnp.int32),
            [pltpu.VMEM((CHUNK, 2 * HID), jnp.float32) for _ in range(KBUF_S)],
            pltpu.VMEM_SHARED((N_PAD, 2 * HID), jnp.float32),
            pltpu.SemaphoreType.DMA,
            pltpu.SemaphoreType.DMA,
        ],
    )
    def _sc_gather_k(h_hbm, idx_hbm, out_hbm, idx_v, rows, table, sem_g,
                     sem_w):
        sid = lax.axis_index("s")
        wid = sid * NC + lax.axis_index("c")
        base = wid * (CHUNK * CPW)
        @pl.when(sid < NS - 1)
        def _():
            pltpu.sync_copy(h_hbm.at[pl.ds(sid * NPS, NPS)],
                            table.at[pl.ds(sid * NPS, NPS)])

        @pl.when(sid == NS - 1)
        def _():
            pltpu.sync_copy(h_hbm.at[pl.ds((NS - 1) * NPS, N_NODES - (NS - 1) * NPS)],
                            table.at[pl.ds((NS - 1) * NPS, N_NODES - (NS - 1) * NPS)])

        pltpu.sync_copy(idx_hbm.at[pl.ds(base, CHUNK * CPW)], idx_v)
        plsc.subcore_barrier()

        def body(j, carry):
            gs = [
                pltpu.async_copy(
                    table.at[idx_v.at[pl.ds((j * KBUF_S + b) * CHUNK, CHUNK)]],
                    rows[b], sem_g)
                for b in range(KBUF_S)
            ]
            for g in gs:
                g.wait()
            ws = [
                pltpu.async_copy(
                    rows[b],
                    out_hbm.at[pl.ds(base + (j * KBUF_S + b) * CHUNK, CHUNK)],
                    sem_w)
                for b in range(KBUF_S)
            ]
            for w in ws:
                w.wait()
            return carry

        lax.fori_loop(0, CPW // KBUF_S, body, 0)

    return _sc_gather_k


def _sc_gather(h, gat_idx):
    return _build_sc_gather()(h, gat_idx)


@functools.cache
def _build_sc_scatter():
    mesh = plsc.VectorSubcoreMesh(core_axis_name="c", subcore_axis_name="s")

    @functools.partial(
        pl.kernel,
        out_type=jax.ShapeDtypeStruct((NC, N_PAD, 2 * HID), jnp.float32),
        mesh=mesh,
        scratch_types=[
            pltpu.VMEM((CHUNK * SCPW,), jnp.int32),
            [pltpu.VMEM((CHUNK, 2 * HID), jnp.float32) for _ in range(KBUF_S)],
            pltpu.VMEM_SHARED((N_PAD, 2 * HID), jnp.float32),
            pltpu.SemaphoreType.DMA,
            pltpu.SemaphoreType.DMA,
        ],
    )
    def _sc_scatter_k(v_hbm, idx_hbm, zeros_hbm, out_hbm, idx_v, vals,
                      table, sem_v, sem_s):
        cid = lax.axis_index("c")
        sid = lax.axis_index("s")
        wid = sid * NC + cid
        pltpu.sync_copy(
            zeros_hbm.at[pl.ds(sid * NPS, NPS)], table.at[pl.ds(sid * NPS, NPS)]
        )
        plsc.subcore_barrier()
        nj = jnp.minimum(SCPW, jnp.maximum(0, NCHUNK - wid * SCPW))
        base = wid * SCPW * CHUNK
        pltpu.sync_copy(idx_hbm.at[pl.ds(base, SCPW * CHUNK)], idx_v)

        def group_body(j, carry):
            ls = [
                pltpu.async_copy(
                    v_hbm.at[pl.ds(base + (j * KBUF_S + b) * CHUNK, CHUNK)],
                    vals[b], sem_v)
                for b in range(KBUF_S)
            ]
            for l in ls:
                l.wait()
            ss = [
                pltpu.async_copy(
                    vals[b],
                    table.at[idx_v.at[pl.ds((j * KBUF_S + b) * CHUNK, CHUNK)]],
                    sem_s, add=True)
                for b in range(KBUF_S)
            ]
            for s in ss:
                s.wait()
            return carry

        lax.fori_loop(0, nj // KBUF_S, group_body, 0)

        def tail_body(j, carry):
            pltpu.async_copy(
                v_hbm.at[pl.ds(base + j * CHUNK, CHUNK)], vals[0], sem_v
            ).wait()
            pltpu.async_copy(
                vals[0], table.at[idx_v.at[pl.ds(j * CHUNK, CHUNK)]],
                sem_s, add=True
            ).wait()
            return carry

        lax.fori_loop((nj // KBUF_S) * KBUF_S, nj, tail_body, 0)

        plsc.subcore_barrier()
        pltpu.sync_copy(
            table.at[pl.ds(sid * NPS, NPS)],
            out_hbm.at[cid, pl.ds(sid * NPS, NPS)],
        )

    return _sc_scatter_k


def _sc_scatter(v_flat, idx2d, zeros_tab):
    return _build_sc_scatter()(v_flat, idx2d, zeros_tab)


def _edge_body(hr0, hr1, hr2, hr3, hc0, hc1, hc2, hc3, ea0, ea1, ea2, ea3,
               we_ref, be_ref, wpad_ref, wb_ref, ai_ref, aj_ref, r4_ref,
               o_ref):
    hrs = (hr0, hr1, hr2, hr3)
    hcs = (hc0, hc1, hc2, hc3)
    eas = (ea0, ea1, ea2, ea3)
    we = we_ref[...]
    be = be_ref[...]
    wpad = wpad_ref[...]
    wb = wb_ref[...]
    ai = ai_ref[...]
    aj = aj_ref[...]
    pooled = None
    for c in range(4):
        e_c = _leaky(
            jnp.dot(eas[c][...], we, preferred_element_type=jnp.float32) + be
        )
        ew = jnp.dot(e_c, wb, preferred_element_type=jnp.float32)
        pi = jnp.dot(hrs[c][...], wpad, preferred_element_type=jnp.float32) + ew
        pj = jnp.dot(hcs[c][...], wpad, preferred_element_type=jnp.float32) + ew
        L = (jnp.dot(pi, ai, preferred_element_type=jnp.float32)
             + jnp.dot(pj, aj, preferred_element_type=jnp.float32)) * LK_A \
            + (jnp.dot(jnp.abs(pi), ai, preferred_element_type=jnp.float32)
               + jnp.dot(jnp.abs(pj), aj,
                         preferred_element_type=jnp.float32)) * LK_B
        L = _leaky(L[:, :HEADS]) * BN_SCALE
        m = jnp.max(L, axis=1, keepdims=True)
        ex = jnp.exp(L - m)
        tot = jnp.sum(ex, axis=1, keepdims=True)
        w4 = ex * (0.25 / tot)
        wrep = jnp.dot(w4, r4_ref[...], preferred_element_type=jnp.float32)
        xj = _leaky(pj)
        contrib = xj * wrep
        pooled = contrib if pooled is None else pooled + contrib
    for hh in range(HEADS):
        o_ref[hh, :, :HID] = pooled[:, hh * HID:(hh + 1) * HID]


def _edge_layer(gathered, ea_perm, we, be, wpad, wb, ai, aj, r4):
    hb = (BQ, 2 * HID)
    spec_h = [
        pl.BlockSpec(hb, functools.partial(lambda c, t: (c * NT + t, 0), c))
        for c in range(4)
    ]
    spec_hc = [
        pl.BlockSpec(
            hb, functools.partial(lambda c, t: (4 * NT + c * NT + t, 0), c)
        )
        for c in range(4)
    ]
    spec_ea = [
        pl.BlockSpec(
            (BQ, F_EDGE), functools.partial(lambda c, t: (c * NT + t, 0), c)
        )
        for c in range(4)
    ]
    full = lambda shape: pl.BlockSpec(shape, lambda t: (0,) * len(shape))
    return pl.pallas_call(
        _edge_body,
        grid=(NT,),
        in_specs=spec_h + spec_hc + spec_ea + [
            full((F_EDGE, HID)), full((1, HID)), full((2 * HID, HEADS * HID)),
            full((HID, HEADS * HID)), full((HEADS * HID, 8)),
            full((HEADS * HID, 8)), full((HEADS, HEADS * HID)),
        ],
        out_specs=pl.BlockSpec((HEADS, BQ, 2 * HID), lambda t: (0, t, 0)),
        out_shape=jax.ShapeDtypeStruct((HEADS, Q, 2 * HID), jnp.float32),
    )(*([gathered] * 8), *([ea_perm] * 4), we, be, wpad, wb, ai, aj, r4)


def _combine0_body(p_ref, b_ref, o_ref):
    r = p_ref[0, :N_NODES, :HID] + p_ref[1, :N_NODES, :HID] + b_ref[...]
    o_ref[...] = jnp.concatenate([r, jnp.zeros_like(r)], axis=1)


def _combine_body(base_ref, p_ref, b_ref, o_ref):
    r = base_ref[...][:, :HID] + p_ref[0, :N_NODES, :HID] \
        + p_ref[1, :N_NODES, :HID] + b_ref[...]
    o_ref[...] = jnp.concatenate([r, jnp.zeros_like(r)], axis=1)


def _combine(partials, bias_row, base=None):
    shape = jax.ShapeDtypeStruct((N_NODES, 2 * HID), jnp.float32)
    if base is None:
        return pl.pallas_call(_combine0_body, out_shape=shape)(partials, bias_row)
    return pl.pallas_call(_combine_body, out_shape=shape)(base, partials, bias_row)


def _readout_body(h_ref, h0_ref, bid_ref, gf_ref, wg1b_ref, wg1a_ref, bg1_ref,
                  wg2_ref, bg2_ref, wo1_ref, bo1_ref, wo2_ref, bo2_ref, o_ref):
    h = h_ref[...][:, :HID] + h0_ref[...][:, :HID]
    bid = bid_ref[...]
    iota = lax.broadcasted_iota(jnp.int32, (N_NODES, N_GRAPHS), 1)
    oh = iota == bid
    ohf = oh.astype(jnp.float32)
    gfw = jnp.dot(gf_ref[...], wg1b_ref[...],
                  preferred_element_type=jnp.float32)
    a = _leaky(
        jnp.dot(h, wg1a_ref[...], preferred_element_type=jnp.float32)
        + jnp.dot(ohf, gfw, preferred_element_type=jnp.float32)
        + bg1_ref[...]
    )
    scores = jnp.dot(a, wg2_ref[...], preferred_element_type=jnp.float32) \
        + bg2_ref[...]
    masked = jnp.where(oh, scores, -1e30)
    m = jnp.max(masked, axis=0, keepdims=True)
    m_sel = jnp.sum(ohf * m, axis=1, keepdims=True)
    ex = jnp.exp(scores - m_sel)
    s = jnp.sum(ohf * ex, axis=0, keepdims=True)
    s_sel = jnp.sum(ohf * s, axis=1, keepdims=True)
    hw = h * (ex / s_sel)
    pooled = lax.dot_general(ohf, hw, (((0,), (0,)), ((), ())),
                             preferred_element_type=jnp.float32)
    z = jnp.maximum(
        jnp.dot(pooled, wo1_ref[...], preferred_element_type=jnp.float32)
        + bo1_ref[...], 0.0)
    o_ref[...] = jnp.dot(z, wo2_ref[...], preferred_element_type=jnp.float32) \
        + bo2_ref[...]


def _readout(h, h0, bid2d, gf, wg1b, wg1a, bg1, wg2, bg2, wo1, bo1, wo2, bo2):
    return pl.pallas_call(
        _readout_body,
        out_shape=jax.ShapeDtypeStruct((N_GRAPHS, 1), jnp.float32),
    )(h, h0, bid2d, gf, wg1b, wg1a, bg1, wg2, bg2, wo1, bo1, wo2, bo2)


def kernel(x, edge_index, edge_attr, batch_idx, global_features, W_node,
           b_node, W_edge, b_edge, conv_W, conv_att, conv_bias, Wg1, bg1,
           Wg2, bg2, Wo1, bo1, Wo2, bo2):
    row = edge_index[0]
    col = edge_index[1]
    row_p = row.reshape(Q, 4).T.reshape(-1)
    col_p = col.reshape(Q, 4).T.reshape(-1)
    gat_idx = jnp.concatenate(
        [row_p, col_p, jnp.zeros((GE_PAD - GE,), jnp.int32)])
    idx_flat = jnp.pad(row, (0, NW * SCPW * CHUNK - E))
    ea_perm = edge_attr.reshape(Q, 4, F_EDGE).transpose(1, 0, 2) \
        .reshape(4 * Q, F_EDGE)
    zeros_tab = jnp.zeros((N_PAD, 2 * HID), jnp.float32)
    bid2d = batch_idx.reshape(N_NODES, 1)
    be_row = b_edge.reshape(1, HID)

    h0 = _node_encoder(x, jnp.pad(W_node, ((0, 0), (0, HID))),
                       jnp.pad(b_node, (0, HID)).reshape(1, 2 * HID))
    h = h0
    for i in range(N_LAYERS):
        gathered = _sc_gather(h, gat_idx)
        att_i = conv_att[i, 0, :, :HID]
        att_j = conv_att[i, 0, :, HID:]
        eye4 = jnp.eye(HEADS, dtype=jnp.float32)
        ai = jnp.pad((eye4[:, None, :] * att_i[:, :, None])
                     .reshape(HEADS * HID, HEADS), ((0, 0), (0, 4)))
        aj = jnp.pad((eye4[:, None, :] * att_j[:, :, None])
                     .reshape(HEADS * HID, HEADS), ((0, 0), (0, 4)))
        wpad = jnp.pad(conv_W[i][:HID], ((0, HID), (0, 0)))
        r4 = (jnp.eye(HEADS, dtype=jnp.float32)[:, :, None]
              * jnp.ones((HID,), jnp.float32)).reshape(HEADS, HEADS * HID)
        v = _edge_layer(gathered, ea_perm, W_edge, be_row, wpad,
                        conv_W[i][HID:], ai, aj, r4)
        partials = _sc_scatter(v.reshape(E, 2 * HID), idx_flat, zeros_tab)
        h = _combine(partials, conv_bias[i].reshape(1, HID),
                     base=None if i == 0 else h)

    out = _readout(
        h, h0, bid2d, global_features, Wg1[HID:], Wg1[:HID],
        bg1.reshape(1, HID), Wg2, bg2.reshape(1, 1), Wo1,
        bo1.reshape(1, HID), Wo2, bo2.reshape(1, 1))
    return out[:, 0]
